# baseline (speedup 1.0000x reference)
import jax
import jax.numpy as jnp
from jax import lax
from jax.experimental import pallas as pl
from jax.experimental.pallas import tpu as pltpu


def _shift_right(a):
    z = jnp.zeros_like(a[:, :1])
    return jnp.concatenate([z, a[:, :-1]], axis=1)


def _shift_left(a):
    z = jnp.zeros_like(a[:, :1])
    return jnp.concatenate([a[:, 1:], z], axis=1)


def _conv_block_kernel(x_ref, w1_ref, b1_ref, w2_ref, b2_ref, out_ref):
    nb = x_ref.shape[0]
    c_out = b2_ref.shape[0]

    w1 = w1_ref[...]
    w2 = w2_ref[...]
    b1 = b1_ref[...]
    b2 = b2_ref[...]

    for i in range(nb):
        x = x_ref[i].astype(jnp.bfloat16)
        xcat = jnp.concatenate(
            [_shift_right(x), x, _shift_left(x)], axis=0)
        h = lax.dot_general(
            w1, xcat, (((1,), (0,)), ((), ())),
            preferred_element_type=jnp.float32)
        h = jnp.maximum(h + b1, 0.0).astype(jnp.bfloat16)
        y = lax.dot_general(
            w2, h, (((1,), (0,)), ((), ())),
            preferred_element_type=jnp.float32)
        acc = y[c_out:2 * c_out] + b2
        acc = acc + _shift_right(y[:c_out])
        acc = acc + _shift_left(y[2 * c_out:])
        out_ref[i] = jnp.maximum(acc, 0.0).astype(out_ref.dtype)


def kernel(x, w1, b1, w2, b2):
    N, C_in, L = x.shape
    C_mid = w1.shape[0]
    C_out = w2.shape[0]

    w1_cat = jnp.concatenate(
        [w1[:, :, 0], w1[:, :, 1], w1[:, :, 2]], axis=1).astype(jnp.bfloat16)
    w2_stk = jnp.concatenate(
        [w2[:, :, 0], w2[:, :, 1], w2[:, :, 2]], axis=0).astype(jnp.bfloat16)
    b1_c = b1.reshape(C_mid, 1).astype(jnp.float32)
    b2_c = b2.reshape(C_out, 1).astype(jnp.float32)

    nb = 32
    while N % nb:
        nb //= 2
    grid = (N // nb,)

    return pl.pallas_call(
        _conv_block_kernel,
        out_shape=jax.ShapeDtypeStruct((N, C_out, L), x.dtype),
        grid=grid,
        in_specs=[
            pl.BlockSpec((nb, C_in, L), lambda b: (0, 0, 0)),
            pl.BlockSpec((C_mid, 3 * C_in), lambda b: (0, 0)),
            pl.BlockSpec((C_mid, 1), lambda b: (0, 0)),
            pl.BlockSpec((3 * C_out, C_mid), lambda b: (0, 0)),
            pl.BlockSpec((C_out, 1), lambda b: (0, 0)),
        ],
        out_specs=pl.BlockSpec((nb, C_out, L), lambda b: (b, 0, 0)),
        compiler_params=pltpu.CompilerParams(
            dimension_semantics=("arbitrary",),
            vmem_limit_bytes=48 * 1024 * 1024,
        ),
    )(x, w1_cat, b1_c, w2_stk, b2_c)


# speedup vs baseline: 1.0020x; 1.0020x over previous
import jax
import jax.numpy as jnp
from jax import lax
from jax.experimental import pallas as pl
from jax.experimental.pallas import tpu as pltpu


def _shift_right(a):
    z = jnp.zeros_like(a[:, :1])
    return jnp.concatenate([z, a[:, :-1]], axis=1)


def _shift_left(a):
    z = jnp.zeros_like(a[:, :1])
    return jnp.concatenate([a[:, 1:], z], axis=1)


def _conv_block_kernel(x_ref, w1_ref, b1_ref, w2_ref, b2_ref, out_ref):
    nb = x_ref.shape[0]
    c_out = b2_ref.shape[0]

    w1 = w1_ref[...]
    w2 = w2_ref[...]
    b1 = b1_ref[...]
    b2 = b2_ref[...]

    for i in range(nb):
        x = x_ref[i].astype(jnp.bfloat16)
        xcat = jnp.concatenate(
            [_shift_right(x), x, _shift_left(x)], axis=0)
        h = lax.dot_general(
            w1, xcat, (((1,), (0,)), ((), ())),
            preferred_element_type=jnp.float32)
        h = jnp.maximum(h + b1, 0.0).astype(jnp.bfloat16)
        y = lax.dot_general(
            w2, h, (((1,), (0,)), ((), ())),
            preferred_element_type=jnp.float32)
        acc = y[c_out:2 * c_out] + b2
        acc = acc + _shift_right(y[:c_out])
        acc = acc + _shift_left(y[2 * c_out:])
        out_ref[i] = jnp.maximum(acc, 0.0).astype(out_ref.dtype)


def kernel(x, w1, b1, w2, b2):
    N, C_in, L = x.shape
    C_mid = w1.shape[0]
    C_out = w2.shape[0]

    w1_cat = jnp.concatenate(
        [w1[:, :, 0], w1[:, :, 1], w1[:, :, 2]], axis=1).astype(jnp.bfloat16)
    w2_stk = jnp.concatenate(
        [w2[:, :, 0], w2[:, :, 1], w2[:, :, 2]], axis=0).astype(jnp.bfloat16)
    b1_c = b1.reshape(C_mid, 1).astype(jnp.float32)
    b2_c = b2.reshape(C_out, 1).astype(jnp.float32)

    nb = 32
    while N % nb:
        nb //= 2
    grid = (N // nb,)

    return pl.pallas_call(
        _conv_block_kernel,
        out_shape=jax.ShapeDtypeStruct((N, C_out, L), x.dtype),
        grid=grid,
        in_specs=[
            pl.BlockSpec((nb, C_in, L), lambda b: (b, 0, 0)),
            pl.BlockSpec((C_mid, 3 * C_in), lambda b: (0, 0)),
            pl.BlockSpec((C_mid, 1), lambda b: (0, 0)),
            pl.BlockSpec((3 * C_out, C_mid), lambda b: (0, 0)),
            pl.BlockSpec((C_out, 1), lambda b: (0, 0)),
        ],
        out_specs=pl.BlockSpec((nb, C_out, L), lambda b: (b, 0, 0)),
        compiler_params=pltpu.CompilerParams(
            dimension_semantics=("arbitrary",),
            vmem_limit_bytes=48 * 1024 * 1024,
        ),
    )(x, w1_cat, b1_c, w2_stk, b2_c)


# speedup vs baseline: 1.4173x; 1.4145x over previous
import jax
import jax.numpy as jnp
from jax import lax
from jax.experimental import pallas as pl
from jax.experimental.pallas import tpu as pltpu


def _shift_right(a):
    z = jnp.zeros_like(a[:, :1])
    return jnp.concatenate([z, a[:, :-1]], axis=1)


def _shift_left(a):
    z = jnp.zeros_like(a[:, :1])
    return jnp.concatenate([a[:, 1:], z], axis=1)


def _conv_block_kernel(x_ref, w1_ref, b1_ref, w2_ref, b2_ref, out_ref):
    nb = x_ref.shape[0]
    c_out = b2_ref.shape[0]

    w1 = w1_ref[...]
    w2 = w2_ref[...]
    b1 = b1_ref[...]
    b2 = b2_ref[...]

    hs = []
    for i in range(nb):
        x = x_ref[i].astype(jnp.bfloat16)
        xcat = jnp.concatenate(
            [_shift_right(x), x, _shift_left(x)], axis=0)
        h = lax.dot_general(
            w1, xcat, (((1,), (0,)), ((), ())),
            preferred_element_type=jnp.float32)
        hs.append(jnp.maximum(h + b1, 0.0).astype(jnp.bfloat16))
    for i in range(nb):
        y = lax.dot_general(
            w2, hs[i], (((1,), (0,)), ((), ())),
            preferred_element_type=jnp.float32)
        acc = y[c_out:2 * c_out] + b2
        acc = acc + _shift_right(y[:c_out])
        acc = acc + _shift_left(y[2 * c_out:])
        out_ref[i] = jnp.maximum(acc, 0.0).astype(out_ref.dtype)


def kernel(x, w1, b1, w2, b2):
    N, C_in, L = x.shape
    C_mid = w1.shape[0]
    C_out = w2.shape[0]

    w1_cat = jnp.concatenate(
        [w1[:, :, 0], w1[:, :, 1], w1[:, :, 2]], axis=1).astype(jnp.bfloat16)
    w2_stk = jnp.concatenate(
        [w2[:, :, 0], w2[:, :, 1], w2[:, :, 2]], axis=0).astype(jnp.bfloat16)
    b1_c = b1.reshape(C_mid, 1).astype(jnp.float32)
    b2_c = b2.reshape(C_out, 1).astype(jnp.float32)

    nb = 32
    while N % nb:
        nb //= 2
    grid = (N // nb,)

    return pl.pallas_call(
        _conv_block_kernel,
        out_shape=jax.ShapeDtypeStruct((N, C_out, L), x.dtype),
        grid=grid,
        in_specs=[
            pl.BlockSpec((nb, C_in, L), lambda b: (b, 0, 0)),
            pl.BlockSpec((C_mid, 3 * C_in), lambda b: (0, 0)),
            pl.BlockSpec((C_mid, 1), lambda b: (0, 0)),
            pl.BlockSpec((3 * C_out, C_mid), lambda b: (0, 0)),
            pl.BlockSpec((C_out, 1), lambda b: (0, 0)),
        ],
        out_specs=pl.BlockSpec((nb, C_out, L), lambda b: (b, 0, 0)),
        compiler_params=pltpu.CompilerParams(
            dimension_semantics=("arbitrary",),
            vmem_limit_bytes=48 * 1024 * 1024,
        ),
    )(x, w1_cat, b1_c, w2_stk, b2_c)
